# baseline (speedup 1.0000x reference)
import jax
import jax.numpy as jnp
from jax import lax
from jax.experimental import pallas as pl
from jax.experimental.pallas import tpu as pltpu
from jax.experimental.pallas import tpu_sc as plsc

N = 10000
D = 128
NC, NS, LN = 2, 16, 16
NW = NC * NS
NPAD = 10240
NPW = NPAD // NS
EPW = 10240
EPAD = NW * EPW
ECH = 32
EBLK = 2048
CPB = EBLK // ECH
NBLK = EPW // EBLK
NRING = 4
LPW = 3200
LPAD = NW * LPW
LCH2 = 64
NLCH = LPW // LCH2
HPT = N // NS

_f32 = jnp.float32
_i32 = jnp.int32


def _zero16():
    return jnp.zeros((LN,), _f32)


def _seg_mean_body(compute_cnt, feat, srcr, dstr, inv_in, agg_out, inv_out,
                   cnt_st, sidx, didx, r0, r1, r2, r3, rbuf, cnt_loc, ctmp,
                   cacc, s0, s1, s2, s3, acc_sh):
    c = lax.axis_index("c")
    s = lax.axis_index("s")
    w = c * NS + s
    rows = [r0, r1, r2, r3]
    sems = [s0, s1, s2, s3]

    z16 = _zero16()

    def zrb_loop(i, _):
        rbuf[i // 8, pl.ds((i % 8) * LN, LN)] = z16
        return 0
    lax.fori_loop(0, 16 * 8, zrb_loop, 0)

    def zcnt_loop(i, _):
        cnt_loc[pl.ds(i * LN, LN)] = z16
        return 0
    lax.fori_loop(0, NPAD // LN, zcnt_loop, 0)

    def zacc_loop(i, _):
        pltpu.sync_copy(rbuf, acc_sh.at[pl.ds(s * NPW + i * 16, 16)])
        return 0
    lax.fori_loop(0, NPW // 16, zacc_loop, 0)

    plsc.subcore_barrier()

    def fire(cj, u):
        pltpu.async_copy(feat.at[sidx.at[cj]], rows[u], sems[u])

    def drain(cj, u):
        pltpu.make_async_copy(feat.at[sidx.at[cj]], rows[u], sems[u]).wait()

    for b in range(NBLK):
        row0 = w * (EPW // ECH) + b * CPB
        pltpu.sync_copy(srcr.at[pl.ds(row0, CPB)], sidx)
        pltpu.sync_copy(dstr.at[pl.ds(row0, CPB)], didx)
        for u in range(NRING):
            fire(u, u)

        def ring(j, _):
            for u in range(NRING):
                cj = j * NRING + u
                drain(cj, u)
                pltpu.sync_copy(rows[u], acc_sh.at[didx.at[cj]], add=True)

                @pl.when(cj + NRING < CPB)
                def _():
                    fire(cj + NRING, u)
            return 0
        lax.fori_loop(0, CPB // NRING, ring, 0)

    if compute_cnt:
        ones = jnp.ones((LN,), _f32)

        def cnt_chunk(q, _):
            pltpu.sync_copy(dstr.at[pl.ds(s * (2 * EPW // ECH) + q * CPB,
                                          CPB)], didx)

            def cnt_loop(j, _):
                for u in range(ECH // LN):
                    idx = didx[j, pl.ds(u * LN, LN)]
                    plsc.addupdate_scatter(cnt_loc, [idx], ones)
                return 0
            lax.fori_loop(0, CPB, cnt_loop, 0)
            return 0
        lax.fori_loop(0, (2 * EPW) // EBLK, cnt_chunk, 0)
        pltpu.sync_copy(cnt_loc, cnt_st.at[pl.ds((c * NS + s) * NPAD, NPAD)])

    plsc.subcore_barrier()

    if compute_cnt:
        pltpu.sync_copy(cnt_st.at[pl.ds(c * NS * NPAD + s * NPW, NPW)], cacc)

        def merge(t, _):
            pltpu.sync_copy(cnt_st.at[pl.ds(c * NS * NPAD + t * NPAD + s * NPW,
                                            NPW)], ctmp)

            def addv(j, _):
                sl = pl.ds(j * LN, LN)
                cacc[sl] = cacc[sl] + ctmp[sl]
                return 0
            lax.fori_loop(0, NPW // LN, addv, 0)
            return 0
        lax.fori_loop(1, NS, merge, 0)

        def invv(j, _):
            sl = pl.ds(j * LN, LN)
            cacc[sl] = 1.0 / jnp.maximum(cacc[sl], 1.0)
            return 0
        lax.fori_loop(0, NPW // LN, invv, 0)

        @pl.when(c == 0)
        def _():
            pltpu.sync_copy(cacc, inv_out.at[pl.ds(s * NPW, NPW)])
    else:
        pltpu.sync_copy(inv_in.at[pl.ds(s * NPW, NPW)], cacc)

    def scale_block(b, _):
        pltpu.sync_copy(acc_sh.at[pl.ds(s * NPW + b * 16, 16)], rbuf)

        def scale_row(r, _):
            iv = plsc.load_gather(cacc, [jnp.full((LN,), b * 16 + r, _i32)])
            for k in range(D // LN):
                rbuf[r, pl.ds(k * LN, LN)] = rbuf[r, pl.ds(k * LN, LN)] * iv
            return 0
        lax.fori_loop(0, 16, scale_row, 0)
        pltpu.sync_copy(rbuf, agg_out.at[pl.ds(c * NPAD + s * NPW + b * 16, 16)])
        return 0
    lax.fori_loop(0, NPW // 16, scale_block, 0)


def _make_seg_mean(compute_cnt):
    mesh = plsc.VectorSubcoreMesh(core_axis_name="c", subcore_axis_name="s")
    out_type = [jax.ShapeDtypeStruct((NC * NPAD, D), _f32)]
    if compute_cnt:
        out_type.append(jax.ShapeDtypeStruct((NPAD,), _f32))
        out_type.append(jax.ShapeDtypeStruct((NC * NS * NPAD,), _f32))
    scratch = [
        pltpu.VMEM((CPB, ECH), _i32),
        pltpu.VMEM((CPB, ECH), _i32),
        pltpu.VMEM((ECH, D), _f32),
        pltpu.VMEM((ECH, D), _f32),
        pltpu.VMEM((ECH, D), _f32),
        pltpu.VMEM((ECH, D), _f32),
        pltpu.VMEM((16, D), _f32),
        pltpu.VMEM((NPAD,), _f32),
        pltpu.VMEM((NPW,), _f32),
        pltpu.VMEM((NPW,), _f32),
        pltpu.SemaphoreType.DMA,
        pltpu.SemaphoreType.DMA,
        pltpu.SemaphoreType.DMA,
        pltpu.SemaphoreType.DMA,
        pltpu.VMEM_SHARED((NPAD, D), _f32),
    ]
    if compute_cnt:
        def body(feat, srcr, dstr, agg_out, inv_out, cnt_st, *rest):
            _seg_mean_body(True, feat, srcr, dstr, None, agg_out, inv_out,
                           cnt_st, *rest)
    else:
        def body(feat, srcr, dstr, inv_in, agg_out, *rest):
            _seg_mean_body(False, feat, srcr, dstr, inv_in, agg_out, None,
                           None, *rest)
    return pl.kernel(body, out_type=tuple(out_type), mesh=mesh,
                     scratch_types=scratch,
                     compiler_params=pltpu.CompilerParams(
                         needs_layout_passes=False))


def _dots_body(h, ia, ib, out, ia0, ia1, ib0, ib1, a0, a1, b0, b1, predv,
               sa0, sa1, sb0, sb1, h_sh):
    c = lax.axis_index("c")
    s = lax.axis_index("s")
    w = c * NS + s
    riota = lax.iota(_i32, LN)
    iav = [ia0, ia1]
    ibv = [ib0, ib1]
    abuf = [a0, a1]
    bbuf = [b0, b1]
    sa = [sa0, sa1]
    sb = [sb0, sb1]

    @pl.when(s < NS - 1)
    def _():
        pltpu.sync_copy(h.at[pl.ds(s * 640, 640)], h_sh.at[pl.ds(s * 640, 640)])

    @pl.when(s == NS - 1)
    def _():
        pltpu.sync_copy(h.at[pl.ds((NS - 1) * 640, N - (NS - 1) * 640)],
                        h_sh.at[pl.ds((NS - 1) * 640, N - (NS - 1) * 640)])
    plsc.subcore_barrier()

    def load_and_fire(i, p):
        eb = (w * NLCH + i) * LCH2
        pltpu.sync_copy(ia.at[pl.ds(eb, LCH2)], iav[p])
        pltpu.sync_copy(ib.at[pl.ds(eb, LCH2)], ibv[p])
        pltpu.async_copy(h_sh.at[iav[p]], abuf[p], sa[p])
        pltpu.async_copy(h_sh.at[ibv[p]], bbuf[p], sb[p])

    load_and_fire(0, 0)
    for i in range(NLCH):
        p = i % 2
        if i + 1 < NLCH:
            load_and_fire(i + 1, (i + 1) % 2)
        pltpu.make_async_copy(h_sh.at[iav[p]], abuf[p], sa[p]).wait()
        pltpu.make_async_copy(h_sh.at[ibv[p]], bbuf[p], sb[p]).wait()

        def group(g, _, _p=p):
            ridx = g * LN + riota

            def chan(t, acc):
                for u in range(8):
                    ch = t * 8 + u
                    cidx = jnp.full((LN,), ch, _i32)
                    va = plsc.load_gather(abuf[_p], [ridx, cidx])
                    vb = plsc.load_gather(bbuf[_p], [ridx, cidx])
                    acc = acc + va * vb
                return acc
            acc = lax.fori_loop(0, D // 8, chan, _zero16())
            predv[pl.ds(g * LN, LN)] = acc
            return 0
        lax.fori_loop(0, LCH2 // LN, group, 0)
        pltpu.sync_copy(predv, out.at[pl.ds((w * NLCH + i) * LCH2, LCH2)])


def _make_dots():
    mesh = plsc.VectorSubcoreMesh(core_axis_name="c", subcore_axis_name="s")
    scratch = [
        pltpu.VMEM((LCH2,), _i32),
        pltpu.VMEM((LCH2,), _i32),
        pltpu.VMEM((LCH2,), _i32),
        pltpu.VMEM((LCH2,), _i32),
        pltpu.VMEM((LCH2, D), _f32),
        pltpu.VMEM((LCH2, D), _f32),
        pltpu.VMEM((LCH2, D), _f32),
        pltpu.VMEM((LCH2, D), _f32),
        pltpu.VMEM((LCH2,), _f32),
        pltpu.SemaphoreType.DMA,
        pltpu.SemaphoreType.DMA,
        pltpu.SemaphoreType.DMA,
        pltpu.SemaphoreType.DMA,
        pltpu.VMEM_SHARED((N, D), _f32),
    ]
    return pl.kernel(_dots_body, out_type=jax.ShapeDtypeStruct((LPAD,), _f32),
                     mesh=mesh, scratch_types=scratch,
                     compiler_params=pltpu.CompilerParams(
                         needs_layout_passes=False))


def _dense_body(x_ref, aggf_ref, wsrc, bsrc, wdst, bdst, wupd, bupd, gam, bet,
                out_ref):
    x = x_ref[...]
    agg = aggf_ref[0:N, :] + aggf_ref[NPAD:NPAD + N, :]
    wu_t = wupd[0:D, :]
    wu_b = wupd[D:2 * D, :]
    hi = jax.lax.Precision.HIGHEST
    w1 = jnp.dot(wdst[...], wu_t, precision=hi)
    w2 = jnp.dot(wsrc[...], wu_b, precision=hi)
    beff = (jnp.dot(bdst[...], wu_t, precision=hi)
            + jnp.dot(bsrc[...], wu_b, precision=hi) + bupd[...])
    h = jnp.dot(x, w1, precision=hi) + jnp.dot(agg, w2, precision=hi) + beff
    m = jnp.mean(h, axis=0, keepdims=True)
    v = jnp.mean(h * h, axis=0, keepdims=True) - m * m
    hn = (h - m) * jax.lax.rsqrt(v + 1.0) * gam[...] + bet[...]
    out_ref[...] = jnp.where(hn >= 0, hn, 0.01 * hn)


def _dense_layer(x, aggf, wsrc, bsrc, wdst, bdst, wupd, bupd, gamma, beta):
    return pl.pallas_call(
        _dense_body,
        out_shape=jax.ShapeDtypeStruct((N, D), _f32),
    )(x, aggf, wsrc, bsrc[None, :], wdst, bdst[None, :], wupd, bupd[None, :],
      gamma[None, :], beta[None, :])


def kernel(x, l1_w_src, l1_b_src, l1_w_dst, l1_b_dst, l1_w_upd, l1_b_upd,
           l2_w_src, l2_b_src, l2_w_dst, l2_b_dst, l2_w_upd, l2_b_upd,
           bn1_gamma, bn1_beta, bn2_gamma, bn2_beta,
           edge_index, edge_label_index):
    E = edge_index.shape[1]
    L = edge_label_index.shape[1]
    src = jnp.concatenate([edge_index[0], jnp.zeros((EPAD - E,), _i32)])
    dst = jnp.concatenate([edge_index[1], jnp.full((EPAD - E,), N, _i32)])
    src = src.reshape(EPAD // ECH, ECH)
    dst = dst.reshape(EPAD // ECH, ECH)
    el0 = jnp.concatenate([edge_label_index[0], jnp.zeros((LPAD - L,), _i32)])
    el1 = jnp.concatenate([edge_label_index[1], jnp.zeros((LPAD - L,), _i32)])

    agg1, inv, _ = _make_seg_mean(True)(x, src, dst)
    h1 = _dense_layer(x, agg1, l1_w_src, l1_b_src, l1_w_dst, l1_b_dst,
                      l1_w_upd, l1_b_upd, bn1_gamma, bn1_beta)
    agg2, = _make_seg_mean(False)(h1, src, dst, inv)
    h2 = _dense_layer(h1, agg2, l2_w_src, l2_b_src, l2_w_dst, l2_b_dst,
                      l2_w_upd, l2_b_upd, bn2_gamma, bn2_beta)
    pred = _make_dots()(h2, el0, el1)
    return pred[:L]


# speedup vs baseline: 1.2836x; 1.2836x over previous
import jax
import jax.numpy as jnp
from jax import lax
from jax.experimental import pallas as pl
from jax.experimental.pallas import tpu as pltpu
from jax.experimental.pallas import tpu_sc as plsc

N = 10000
D = 128
NC, NS, LN = 2, 16, 16
NW = NC * NS
NPAD = 10240
NPW = NPAD // NS
EPW = 10240
EPAD = NW * EPW
ECH = 32
EBLK = 2048
CPB = EBLK // ECH
NBLK = EPW // EBLK
NRING = 4
LPW = 3200
LPAD = NW * LPW
LCH2 = 64
NLCH = LPW // LCH2
HPT = N // NS

_f32 = jnp.float32
_i32 = jnp.int32


def _zero16():
    return jnp.zeros((LN,), _f32)


def _seg_mean_body(compute_cnt, feat, srcr, dstr, inv_in, agg_out, inv_out,
                   cnt_st, sidx, didx, r0, r1, r2, r3, rbuf, cnt_loc, ctmp,
                   cacc, s0, s1, s2, s3, acc_sh):
    c = lax.axis_index("c")
    s = lax.axis_index("s")
    w = c * NS + s
    rows = [r0, r1, r2, r3]
    sems = [s0, s1, s2, s3]

    z16 = _zero16()

    def zrb_loop(i, _):
        rbuf[i // 8, pl.ds((i % 8) * LN, LN)] = z16
        return 0
    lax.fori_loop(0, 16 * 8, zrb_loop, 0)

    def zcnt_loop(i, _):
        cnt_loc[pl.ds(i * LN, LN)] = z16
        return 0
    lax.fori_loop(0, NPAD // LN, zcnt_loop, 0)

    def zacc_loop(i, _):
        pltpu.sync_copy(rbuf, acc_sh.at[pl.ds(s * NPW + i * 16, 16)])
        return 0
    lax.fori_loop(0, NPW // 16, zacc_loop, 0)

    plsc.subcore_barrier()

    def fire(cj, u):
        pltpu.async_copy(feat.at[sidx.at[cj]], rows[u], sems[u])

    def drain(cj, u):
        pltpu.make_async_copy(feat.at[sidx.at[cj]], rows[u], sems[u]).wait()

    for b in range(NBLK):
        row0 = w * (EPW // ECH) + b * CPB
        pltpu.sync_copy(srcr.at[pl.ds(row0, CPB)], sidx)
        pltpu.sync_copy(dstr.at[pl.ds(row0, CPB)], didx)
        for u in range(NRING):
            fire(u, u)

        def ring(j, _):
            for u in range(NRING):
                cj = j * NRING + u
                drain(cj, u)
                pltpu.sync_copy(rows[u], acc_sh.at[didx.at[cj]], add=True)

                @pl.when(cj + NRING < CPB)
                def _():
                    fire(cj + NRING, u)
            return 0
        lax.fori_loop(0, CPB // NRING, ring, 0)

    if compute_cnt:
        ones = jnp.ones((LN,), _f32)

        def cnt_chunk(q, _):
            pltpu.sync_copy(dstr.at[pl.ds(s * (2 * EPW // ECH) + q * CPB,
                                          CPB)], didx)

            def cnt_loop(j, _):
                for u in range(ECH // LN):
                    idx = didx[j, pl.ds(u * LN, LN)]
                    plsc.addupdate_scatter(cnt_loc, [idx], ones)
                return 0
            lax.fori_loop(0, CPB, cnt_loop, 0)
            return 0
        lax.fori_loop(0, (2 * EPW) // EBLK, cnt_chunk, 0)
        pltpu.sync_copy(cnt_loc, cnt_st.at[pl.ds((c * NS + s) * NPAD, NPAD)])

    plsc.subcore_barrier()

    if compute_cnt:
        pltpu.sync_copy(cnt_st.at[pl.ds(c * NS * NPAD + s * NPW, NPW)], cacc)

        def merge(t, _):
            pltpu.sync_copy(cnt_st.at[pl.ds(c * NS * NPAD + t * NPAD + s * NPW,
                                            NPW)], ctmp)

            def addv(j, _):
                sl = pl.ds(j * LN, LN)
                cacc[sl] = cacc[sl] + ctmp[sl]
                return 0
            lax.fori_loop(0, NPW // LN, addv, 0)
            return 0
        lax.fori_loop(1, NS, merge, 0)

        def invv(j, _):
            sl = pl.ds(j * LN, LN)
            cacc[sl] = 1.0 / jnp.maximum(cacc[sl], 1.0)
            return 0
        lax.fori_loop(0, NPW // LN, invv, 0)

        @pl.when(c == 0)
        def _():
            pltpu.sync_copy(cacc, inv_out.at[pl.ds(s * NPW, NPW)])
    else:
        pltpu.sync_copy(inv_in.at[pl.ds(s * NPW, NPW)], cacc)

    def scale_block(b, _):
        pltpu.sync_copy(acc_sh.at[pl.ds(s * NPW + b * 16, 16)], rbuf)

        def scale_row(r, _):
            iv = plsc.load_gather(cacc, [jnp.full((LN,), b * 16 + r, _i32)])
            for k in range(D // LN):
                rbuf[r, pl.ds(k * LN, LN)] = rbuf[r, pl.ds(k * LN, LN)] * iv
            return 0
        lax.fori_loop(0, 16, scale_row, 0)
        pltpu.sync_copy(rbuf, agg_out.at[pl.ds(c * NPAD + s * NPW + b * 16, 16)])
        return 0
    lax.fori_loop(0, NPW // 16, scale_block, 0)


def _make_seg_mean(compute_cnt):
    mesh = plsc.VectorSubcoreMesh(core_axis_name="c", subcore_axis_name="s")
    out_type = [jax.ShapeDtypeStruct((NC * NPAD, D), _f32)]
    if compute_cnt:
        out_type.append(jax.ShapeDtypeStruct((NPAD,), _f32))
        out_type.append(jax.ShapeDtypeStruct((NC * NS * NPAD,), _f32))
    scratch = [
        pltpu.VMEM((CPB, ECH), _i32),
        pltpu.VMEM((CPB, ECH), _i32),
        pltpu.VMEM((ECH, D), _f32),
        pltpu.VMEM((ECH, D), _f32),
        pltpu.VMEM((ECH, D), _f32),
        pltpu.VMEM((ECH, D), _f32),
        pltpu.VMEM((16, D), _f32),
        pltpu.VMEM((NPAD,), _f32),
        pltpu.VMEM((NPW,), _f32),
        pltpu.VMEM((NPW,), _f32),
        pltpu.SemaphoreType.DMA,
        pltpu.SemaphoreType.DMA,
        pltpu.SemaphoreType.DMA,
        pltpu.SemaphoreType.DMA,
        pltpu.VMEM_SHARED((NPAD, D), _f32),
    ]
    if compute_cnt:
        def body(feat, srcr, dstr, agg_out, inv_out, cnt_st, *rest):
            _seg_mean_body(True, feat, srcr, dstr, None, agg_out, inv_out,
                           cnt_st, *rest)
    else:
        def body(feat, srcr, dstr, inv_in, agg_out, *rest):
            _seg_mean_body(False, feat, srcr, dstr, inv_in, agg_out, None,
                           None, *rest)
    return pl.kernel(body, out_type=tuple(out_type), mesh=mesh,
                     scratch_types=scratch,
                     compiler_params=pltpu.CompilerParams(
                         needs_layout_passes=False))


def _dots_body(h, ia, ib, out, ia0, ia1, ib0, ib1, a0, a1, b0, b1, predv,
               tbuf, sa0, sa1, sb0, sb1, h_sh):
    c = lax.axis_index("c")
    s = lax.axis_index("s")
    w = c * NS + s
    riota = lax.iota(_i32, LN)
    iav = [ia0, ia1]
    ibv = [ib0, ib1]
    abuf = [a0, a1]
    bbuf = [b0, b1]
    sa = [sa0, sa1]
    sb = [sb0, sb1]

    @pl.when(s < NS - 1)
    def _():
        pltpu.sync_copy(h.at[pl.ds(s * 640, 640)], h_sh.at[pl.ds(s * 640, 640)])

    @pl.when(s == NS - 1)
    def _():
        pltpu.sync_copy(h.at[pl.ds((NS - 1) * 640, N - (NS - 1) * 640)],
                        h_sh.at[pl.ds((NS - 1) * 640, N - (NS - 1) * 640)])
    plsc.subcore_barrier()

    def load_and_fire(i, p):
        eb = (w * NLCH + i) * LCH2
        pltpu.sync_copy(ia.at[pl.ds(eb, LCH2)], iav[p])
        pltpu.sync_copy(ib.at[pl.ds(eb, LCH2)], ibv[p])
        pltpu.async_copy(h_sh.at[iav[p]], abuf[p], sa[p])
        pltpu.async_copy(h_sh.at[ibv[p]], bbuf[p], sb[p])

    load_and_fire(0, 0)
    for i in range(NLCH):
        p = i % 2
        if i + 1 < NLCH:
            load_and_fire(i + 1, (i + 1) % 2)
        pltpu.make_async_copy(h_sh.at[iav[p]], abuf[p], sa[p]).wait()
        pltpu.make_async_copy(h_sh.at[ibv[p]], bbuf[p], sb[p]).wait()

        def group(g, _, _p=p):
            def edge2(e2, _):
                for q in range(2):
                    e = g * LN + e2 * 2 + q
                    acc = (abuf[_p][e, pl.ds(0, LN)]
                           * bbuf[_p][e, pl.ds(0, LN)])
                    for k in range(1, D // LN):
                        acc = acc + (abuf[_p][e, pl.ds(k * LN, LN)]
                                     * bbuf[_p][e, pl.ds(k * LN, LN)])
                    tbuf[pl.ds((e2 * 2 + q) * 17, LN)] = acc
                return 0
            lax.fori_loop(0, LN // 2, edge2, 0)

            def colsum(cc, a):
                return a + plsc.load_gather(tbuf, [riota * 17 + cc])
            acc2 = lax.fori_loop(0, LN, colsum, _zero16())
            predv[pl.ds(g * LN, LN)] = acc2
            return 0
        lax.fori_loop(0, LCH2 // LN, group, 0)
        pltpu.sync_copy(predv, out.at[pl.ds((w * NLCH + i) * LCH2, LCH2)])


def _make_dots():
    mesh = plsc.VectorSubcoreMesh(core_axis_name="c", subcore_axis_name="s")
    scratch = [
        pltpu.VMEM((LCH2,), _i32),
        pltpu.VMEM((LCH2,), _i32),
        pltpu.VMEM((LCH2,), _i32),
        pltpu.VMEM((LCH2,), _i32),
        pltpu.VMEM((LCH2, D), _f32),
        pltpu.VMEM((LCH2, D), _f32),
        pltpu.VMEM((LCH2, D), _f32),
        pltpu.VMEM((LCH2, D), _f32),
        pltpu.VMEM((LCH2,), _f32),
        pltpu.VMEM((LN * 17, ), _f32),
        pltpu.SemaphoreType.DMA,
        pltpu.SemaphoreType.DMA,
        pltpu.SemaphoreType.DMA,
        pltpu.SemaphoreType.DMA,
        pltpu.VMEM_SHARED((N, D), _f32),
    ]
    return pl.kernel(_dots_body, out_type=jax.ShapeDtypeStruct((LPAD,), _f32),
                     mesh=mesh, scratch_types=scratch,
                     compiler_params=pltpu.CompilerParams(
                         needs_layout_passes=False))


def _dense_body(x_ref, aggf_ref, wsrc, bsrc, wdst, bdst, wupd, bupd, gam, bet,
                out_ref):
    x = x_ref[...]
    agg = aggf_ref[0:N, :] + aggf_ref[NPAD:NPAD + N, :]
    wu_t = wupd[0:D, :]
    wu_b = wupd[D:2 * D, :]
    hi = jax.lax.Precision.HIGHEST
    w1 = jnp.dot(wdst[...], wu_t, precision=hi)
    w2 = jnp.dot(wsrc[...], wu_b, precision=hi)
    beff = (jnp.dot(bdst[...], wu_t, precision=hi)
            + jnp.dot(bsrc[...], wu_b, precision=hi) + bupd[...])
    h = jnp.dot(x, w1, precision=hi) + jnp.dot(agg, w2, precision=hi) + beff
    m = jnp.mean(h, axis=0, keepdims=True)
    v = jnp.mean(h * h, axis=0, keepdims=True) - m * m
    hn = (h - m) * jax.lax.rsqrt(v + 1.0) * gam[...] + bet[...]
    out_ref[...] = jnp.where(hn >= 0, hn, 0.01 * hn)


def _dense_layer(x, aggf, wsrc, bsrc, wdst, bdst, wupd, bupd, gamma, beta):
    return pl.pallas_call(
        _dense_body,
        out_shape=jax.ShapeDtypeStruct((N, D), _f32),
    )(x, aggf, wsrc, bsrc[None, :], wdst, bdst[None, :], wupd, bupd[None, :],
      gamma[None, :], beta[None, :])


def kernel(x, l1_w_src, l1_b_src, l1_w_dst, l1_b_dst, l1_w_upd, l1_b_upd,
           l2_w_src, l2_b_src, l2_w_dst, l2_b_dst, l2_w_upd, l2_b_upd,
           bn1_gamma, bn1_beta, bn2_gamma, bn2_beta,
           edge_index, edge_label_index):
    E = edge_index.shape[1]
    L = edge_label_index.shape[1]
    src = jnp.concatenate([edge_index[0], jnp.zeros((EPAD - E,), _i32)])
    dst = jnp.concatenate([edge_index[1], jnp.full((EPAD - E,), N, _i32)])
    src = src.reshape(EPAD // ECH, ECH)
    dst = dst.reshape(EPAD // ECH, ECH)
    el0 = jnp.concatenate([edge_label_index[0], jnp.zeros((LPAD - L,), _i32)])
    el1 = jnp.concatenate([edge_label_index[1], jnp.zeros((LPAD - L,), _i32)])

    agg1, inv, _ = _make_seg_mean(True)(x, src, dst)
    h1 = _dense_layer(x, agg1, l1_w_src, l1_b_src, l1_w_dst, l1_b_dst,
                      l1_w_upd, l1_b_upd, bn1_gamma, bn1_beta)
    agg2, = _make_seg_mean(False)(h1, src, dst, inv)
    h2 = _dense_layer(h1, agg2, l2_w_src, l2_b_src, l2_w_dst, l2_b_dst,
                      l2_w_upd, l2_b_upd, bn2_gamma, bn2_beta)
    pred = _make_dots()(h2, el0, el1)
    return pred[:L]


# speedup vs baseline: 1.3438x; 1.0469x over previous
import functools

import jax
import jax.numpy as jnp
from jax import lax
from jax.experimental import pallas as pl
from jax.experimental.pallas import tpu as pltpu
from jax.experimental.pallas import tpu_sc as plsc

N = 10000
D = 128
NC, NS, LN = 2, 16, 16
NW = NC * NS
NPAD = 10240
NPW = NPAD // NS
EPAD = 327680
ECH = 32
EBLK = 2048
CPB = EBLK // ECH
NRING = 4
NBLK_F = 7
NBLK_S = 3
FAST_C = 0
LPW = 3200
LPAD = NW * LPW
LCH2 = 64
NLCH = LPW // LCH2

_f32 = jnp.float32
_i32 = jnp.int32


def _zero16():
    return jnp.zeros((LN,), _f32)


def _seg_mean_body(compute_cnt, feat, srcr, dstr, inv_in, agg_out, inv_out,
                   cnt_st, sidx, didx, r0, r1, r2, r3, rbuf, cnt_loc, ctmp,
                   cacc, s0, s1, s2, s3, acc_sh):
    c = lax.axis_index("c")
    s = lax.axis_index("s")
    rows = [r0, r1, r2, r3]
    sems = [s0, s1, s2, s3]
    fast = c == FAST_C

    z16 = _zero16()

    def zrb_loop(i, _):
        rbuf[i // 8, pl.ds((i % 8) * LN, LN)] = z16
        return 0
    lax.fori_loop(0, 16 * 8, zrb_loop, 0)

    def zcnt_loop(i, _):
        cnt_loc[pl.ds(i * LN, LN)] = z16
        return 0
    lax.fori_loop(0, NPAD // LN, zcnt_loop, 0)

    def zacc_loop(i, _):
        pltpu.sync_copy(rbuf, acc_sh.at[pl.ds(s * NPW + i * 16, 16)])
        return 0
    lax.fori_loop(0, NPW // 16, zacc_loop, 0)

    plsc.subcore_barrier()

    def fire(cj, u):
        pltpu.async_copy(feat.at[sidx.at[cj]], rows[u], sems[u])

    def drain(cj, u):
        pltpu.make_async_copy(feat.at[sidx.at[cj]], rows[u], sems[u]).wait()

    row_f = s * (NBLK_F * CPB)
    row_s = NS * (NBLK_F * CPB) + s * (NBLK_S * CPB)
    base_row = jnp.where(fast, row_f, row_s)

    for b in range(NBLK_F):
        @pl.when(fast if b >= NBLK_S else (c >= 0))
        def _():
            row0 = base_row + b * CPB
            pltpu.sync_copy(srcr.at[pl.ds(row0, CPB)], sidx)
            pltpu.sync_copy(dstr.at[pl.ds(row0, CPB)], didx)
            for u in range(NRING):
                fire(u, u)

            def ring(j, _):
                for u in range(NRING):
                    cj = j * NRING + u
                    drain(cj, u)
                    pltpu.sync_copy(rows[u], acc_sh.at[didx.at[cj]], add=True)

                    @pl.when(cj + NRING < CPB)
                    def _():
                        fire(cj + NRING, u)
                return 0
            lax.fori_loop(0, CPB // NRING, ring, 0)

    if compute_cnt:
        ones = jnp.ones((LN,), _f32)
        ebt = EPAD // NS // ECH

        def cnt_chunk(q, _):
            pltpu.sync_copy(dstr.at[pl.ds(s * ebt + q * CPB, CPB)], didx)

            def cnt_loop(j, _):
                for u in range(ECH // LN):
                    idx = didx[j, pl.ds(u * LN, LN)]
                    plsc.addupdate_scatter(cnt_loc, [idx], ones)
                return 0
            lax.fori_loop(0, CPB, cnt_loop, 0)
            return 0
        lax.fori_loop(0, ebt // CPB, cnt_chunk, 0)
        pltpu.sync_copy(cnt_loc, cnt_st.at[pl.ds((c * NS + s) * NPAD, NPAD)])

    plsc.subcore_barrier()

    if compute_cnt:
        pltpu.sync_copy(cnt_st.at[pl.ds(c * NS * NPAD + s * NPW, NPW)], cacc)

        def merge(t, _):
            pltpu.sync_copy(cnt_st.at[pl.ds(c * NS * NPAD + t * NPAD + s * NPW,
                                            NPW)], ctmp)

            def addv(j, _):
                sl = pl.ds(j * LN, LN)
                cacc[sl] = cacc[sl] + ctmp[sl]
                return 0
            lax.fori_loop(0, NPW // LN, addv, 0)
            return 0
        lax.fori_loop(1, NS, merge, 0)

        def invv(j, _):
            sl = pl.ds(j * LN, LN)
            cacc[sl] = 1.0 / jnp.maximum(cacc[sl], 1.0)
            return 0
        lax.fori_loop(0, NPW // LN, invv, 0)

        @pl.when(c == 0)
        def _():
            pltpu.sync_copy(cacc, inv_out.at[pl.ds(s * NPW, NPW)])
    else:
        pltpu.sync_copy(inv_in.at[pl.ds(s * NPW, NPW)], cacc)

    def scale_block(b, _):
        pltpu.sync_copy(acc_sh.at[pl.ds(s * NPW + b * 16, 16)], rbuf)

        def scale_row(r, _):
            iv = plsc.load_gather(cacc, [jnp.full((LN,), b * 16 + r, _i32)])
            for k in range(D // LN):
                rbuf[r, pl.ds(k * LN, LN)] = rbuf[r, pl.ds(k * LN, LN)] * iv
            return 0
        lax.fori_loop(0, 16, scale_row, 0)
        pltpu.sync_copy(rbuf, agg_out.at[pl.ds(c * NPAD + s * NPW + b * 16, 16)])
        return 0
    lax.fori_loop(0, NPW // 16, scale_block, 0)


def _make_seg_mean(compute_cnt):
    mesh = plsc.VectorSubcoreMesh(core_axis_name="c", subcore_axis_name="s")
    out_type = [jax.ShapeDtypeStruct((NC * NPAD, D), _f32)]
    if compute_cnt:
        out_type.append(jax.ShapeDtypeStruct((NPAD,), _f32))
        out_type.append(jax.ShapeDtypeStruct((NC * NS * NPAD,), _f32))
    scratch = [
        pltpu.VMEM((CPB, ECH), _i32),
        pltpu.VMEM((CPB, ECH), _i32),
        pltpu.VMEM((ECH, D), _f32),
        pltpu.VMEM((ECH, D), _f32),
        pltpu.VMEM((ECH, D), _f32),
        pltpu.VMEM((ECH, D), _f32),
        pltpu.VMEM((16, D), _f32),
        pltpu.VMEM((NPAD,), _f32),
        pltpu.VMEM((NPW,), _f32),
        pltpu.VMEM((NPW,), _f32),
        pltpu.SemaphoreType.DMA,
        pltpu.SemaphoreType.DMA,
        pltpu.SemaphoreType.DMA,
        pltpu.SemaphoreType.DMA,
        pltpu.VMEM_SHARED((NPAD, D), _f32),
    ]
    if compute_cnt:
        def body(feat, srcr, dstr, agg_out, inv_out, cnt_st, *rest):
            _seg_mean_body(True, feat, srcr, dstr, None, agg_out, inv_out,
                           cnt_st, *rest)
    else:
        def body(feat, srcr, dstr, inv_in, agg_out, *rest):
            _seg_mean_body(False, feat, srcr, dstr, inv_in, agg_out, None,
                           None, *rest)
    return pl.kernel(body, out_type=tuple(out_type), mesh=mesh,
                     scratch_types=scratch,
                     compiler_params=pltpu.CompilerParams(
                         needs_layout_passes=False))


def _dots_body(h, ia, ib, out, ia0, ia1, ib0, ib1, a0, a1, b0, b1, predv,
               tbuf, sa0, sa1, sb0, sb1, h_sh):
    c = lax.axis_index("c")
    s = lax.axis_index("s")
    w = c * NS + s
    riota = lax.iota(_i32, LN)
    iav = [ia0, ia1]
    ibv = [ib0, ib1]
    abuf = [a0, a1]
    bbuf = [b0, b1]
    sa = [sa0, sa1]
    sb = [sb0, sb1]

    @pl.when(s < NS - 1)
    def _():
        pltpu.sync_copy(h.at[pl.ds(s * 640, 640)], h_sh.at[pl.ds(s * 640, 640)])

    @pl.when(s == NS - 1)
    def _():
        pltpu.sync_copy(h.at[pl.ds((NS - 1) * 640, N - (NS - 1) * 640)],
                        h_sh.at[pl.ds((NS - 1) * 640, N - (NS - 1) * 640)])
    plsc.subcore_barrier()

    def load_and_fire(i, p):
        eb = (w * NLCH + i) * LCH2
        pltpu.sync_copy(ia.at[pl.ds(eb, LCH2)], iav[p])
        pltpu.sync_copy(ib.at[pl.ds(eb, LCH2)], ibv[p])
        pltpu.async_copy(h_sh.at[iav[p]], abuf[p], sa[p])
        pltpu.async_copy(h_sh.at[ibv[p]], bbuf[p], sb[p])

    load_and_fire(0, 0)
    for i in range(NLCH):
        p = i % 2
        if i + 1 < NLCH:
            load_and_fire(i + 1, (i + 1) % 2)
        pltpu.make_async_copy(h_sh.at[iav[p]], abuf[p], sa[p]).wait()
        pltpu.make_async_copy(h_sh.at[ibv[p]], bbuf[p], sb[p]).wait()

        def group(g, _, _p=p):
            def edge2(e2, _):
                for q in range(2):
                    e = g * LN + e2 * 2 + q
                    acc = (abuf[_p][e, pl.ds(0, LN)]
                           * bbuf[_p][e, pl.ds(0, LN)])
                    for k in range(1, D // LN):
                        acc = acc + (abuf[_p][e, pl.ds(k * LN, LN)]
                                     * bbuf[_p][e, pl.ds(k * LN, LN)])
                    tbuf[pl.ds((e2 * 2 + q) * 17, LN)] = acc
                return 0
            lax.fori_loop(0, LN // 2, edge2, 0)

            def colsum(cc, a):
                return a + plsc.load_gather(tbuf, [riota * 17 + cc])
            acc2 = lax.fori_loop(0, LN, colsum, _zero16())
            predv[pl.ds(g * LN, LN)] = acc2
            return 0
        lax.fori_loop(0, LCH2 // LN, group, 0)
        pltpu.sync_copy(predv, out.at[pl.ds((w * NLCH + i) * LCH2, LCH2)])


def _make_dots():
    mesh = plsc.VectorSubcoreMesh(core_axis_name="c", subcore_axis_name="s")
    scratch = [
        pltpu.VMEM((LCH2,), _i32),
        pltpu.VMEM((LCH2,), _i32),
        pltpu.VMEM((LCH2,), _i32),
        pltpu.VMEM((LCH2,), _i32),
        pltpu.VMEM((LCH2, D), _f32),
        pltpu.VMEM((LCH2, D), _f32),
        pltpu.VMEM((LCH2, D), _f32),
        pltpu.VMEM((LCH2, D), _f32),
        pltpu.VMEM((LCH2,), _f32),
        pltpu.VMEM((LN * 17, ), _f32),
        pltpu.SemaphoreType.DMA,
        pltpu.SemaphoreType.DMA,
        pltpu.SemaphoreType.DMA,
        pltpu.SemaphoreType.DMA,
        pltpu.VMEM_SHARED((N, D), _f32),
    ]
    return pl.kernel(_dots_body, out_type=jax.ShapeDtypeStruct((LPAD,), _f32),
                     mesh=mesh, scratch_types=scratch,
                     compiler_params=pltpu.CompilerParams(
                         needs_layout_passes=False))


def _dense_body(x_ref, aggf_ref, wsrc, bsrc, wdst, bdst, wupd, bupd, gam, bet,
                out_ref):
    x = x_ref[...]
    agg = aggf_ref[0:N, :] + aggf_ref[NPAD:NPAD + N, :]
    wu_t = wupd[0:D, :]
    wu_b = wupd[D:2 * D, :]
    hi = jax.lax.Precision.HIGHEST
    w1 = jnp.dot(wdst[...], wu_t, precision=hi)
    w2 = jnp.dot(wsrc[...], wu_b, precision=hi)
    beff = (jnp.dot(bdst[...], wu_t, precision=hi)
            + jnp.dot(bsrc[...], wu_b, precision=hi) + bupd[...])
    h = jnp.dot(x, w1, precision=hi) + jnp.dot(agg, w2, precision=hi) + beff
    m = jnp.mean(h, axis=0, keepdims=True)
    v = jnp.mean(h * h, axis=0, keepdims=True) - m * m
    hn = (h - m) * jax.lax.rsqrt(v + 1.0) * gam[...] + bet[...]
    out_ref[...] = jnp.where(hn >= 0, hn, 0.01 * hn)


def _dense_layer(x, aggf, wsrc, bsrc, wdst, bdst, wupd, bupd, gamma, beta):
    return pl.pallas_call(
        _dense_body,
        out_shape=jax.ShapeDtypeStruct((N, D), _f32),
    )(x, aggf, wsrc, bsrc[None, :], wdst, bdst[None, :], wupd, bupd[None, :],
      gamma[None, :], beta[None, :])


def kernel(x, l1_w_src, l1_b_src, l1_w_dst, l1_b_dst, l1_w_upd, l1_b_upd,
           l2_w_src, l2_b_src, l2_w_dst, l2_b_dst, l2_w_upd, l2_b_upd,
           bn1_gamma, bn1_beta, bn2_gamma, bn2_beta,
           edge_index, edge_label_index):
    E = edge_index.shape[1]
    L = edge_label_index.shape[1]
    src = jnp.concatenate([edge_index[0], jnp.zeros((EPAD - E,), _i32)])
    dst = jnp.concatenate([edge_index[1], jnp.full((EPAD - E,), N, _i32)])
    el0 = jnp.concatenate([edge_label_index[0], jnp.zeros((LPAD - L,), _i32)])
    el1 = jnp.concatenate([edge_label_index[1], jnp.zeros((LPAD - L,), _i32)])
    src = src.reshape(EPAD // ECH, ECH)
    dst = dst.reshape(EPAD // ECH, ECH)

    agg1, inv, _ = _make_seg_mean(True)(x, src, dst)
    h1 = _dense_layer(x, agg1, l1_w_src, l1_b_src, l1_w_dst, l1_b_dst,
                      l1_w_upd, l1_b_upd, bn1_gamma, bn1_beta)
    agg2, = _make_seg_mean(False)(h1, src, dst, inv)
    h2 = _dense_layer(h1, agg2, l2_w_src, l2_b_src, l2_w_dst, l2_b_dst,
                      l2_w_upd, l2_b_upd, bn2_gamma, bn2_beta)
    pred = _make_dots()(h2, el0, el1)
    return pred[:L]


# speedup vs baseline: 1.4013x; 1.0428x over previous
import functools

import jax
import jax.numpy as jnp
from jax import lax
from jax.experimental import pallas as pl
from jax.experimental.pallas import tpu as pltpu
from jax.experimental.pallas import tpu_sc as plsc

N = 10000
D = 128
NC, NS, LN = 2, 16, 16
NW = NC * NS
NPAD = 10240
NPW = NPAD // NS
EPAD = 327680
ECH = 32
EBLK = 2048
CPB = EBLK // ECH
NRING = 2
NBLK_F = 7
NBLK_S = 3
FAST_C = 0
LPW = 3200
LPAD = NW * LPW
LCH2 = 64
NLCH = LPW // LCH2

_f32 = jnp.float32
_i32 = jnp.int32


def _zero16():
    return jnp.zeros((LN,), _f32)


def _seg_mean_body(compute_cnt, feat, srcr, dstr, inv_in, agg_out, inv_out,
                   cnt_st, sidx, didx, r0, r1, rbuf, rbuf2,
                   cnt_loc, cacc, s0, s1, s2, s3, acc_sh):
    c = lax.axis_index("c")
    s = lax.axis_index("s")
    rows = [r0, r1]
    sems = [s2, s3]
    fast = c == FAST_C

    z16 = _zero16()

    def zrb_loop(i, _):
        rbuf[i // 8, pl.ds((i % 8) * LN, LN)] = z16
        return 0
    lax.fori_loop(0, 16 * 8, zrb_loop, 0)

    def zcnt_loop(i, _):
        cnt_loc[pl.ds(i * LN, LN)] = z16
        return 0
    lax.fori_loop(0, NPAD // LN, zcnt_loop, 0)

    def zacc_loop(i, _):
        pltpu.sync_copy(rbuf, acc_sh.at[pl.ds(s * NPW + i * 16, 16)])
        return 0
    lax.fori_loop(0, NPW // 16, zacc_loop, 0)

    plsc.subcore_barrier()

    def fire(cj, u):
        pltpu.async_copy(feat.at[sidx.at[cj]], rows[u], sems[u])

    def drain(cj, u):
        pltpu.make_async_copy(feat.at[sidx.at[cj]], rows[u], sems[u]).wait()

    row_f = s * (NBLK_F * CPB)
    row_s = NS * (NBLK_F * CPB) + s * (NBLK_S * CPB)
    base_row = jnp.where(fast, row_f, row_s)

    for b in range(NBLK_F):
        @pl.when(fast if b >= NBLK_S else (c >= 0))
        def _():
            row0 = base_row + b * CPB
            pltpu.sync_copy(srcr.at[pl.ds(row0, CPB)], sidx)
            pltpu.sync_copy(dstr.at[pl.ds(row0, CPB)], didx)
            for u in range(NRING):
                fire(u, u)

            def ring(j, _):
                for u in range(NRING):
                    cj = j * NRING + u
                    drain(cj, u)
                    pltpu.sync_copy(rows[u], acc_sh.at[didx.at[cj]], add=True)

                    @pl.when(cj + NRING < CPB)
                    def _():
                        fire(cj + NRING, u)
                return 0
            lax.fori_loop(0, CPB // NRING, ring, 0)

    if compute_cnt:
        ones = jnp.ones((LN,), _f32)
        ebt = EPAD // NS // ECH
        nq = ebt // CPB
        dbufs = [sidx, didx]
        csems = [s2, s3]

        def cfire(q, p):
            pltpu.async_copy(dstr.at[pl.ds(s * ebt + q * CPB, CPB)],
                             dbufs[p], csems[p])

        cfire(0, 0)
        for q in range(nq):
            p = q % 2
            if q + 1 < nq:
                cfire(q + 1, (q + 1) % 2)
            pltpu.make_async_copy(dstr.at[pl.ds(s * ebt + q * CPB, CPB)],
                                  dbufs[p], csems[p]).wait()

            def cnt_loop(j, _, _p=p):
                for u in range(ECH // LN):
                    idx = dbufs[_p][j, pl.ds(u * LN, LN)]
                    plsc.addupdate_scatter(cnt_loc, [idx], ones)
                return 0
            lax.fori_loop(0, CPB, cnt_loop, 0)

        for t in range(NS):
            pltpu.async_copy(cnt_loc.at[pl.ds(t * NPW, NPW)],
                             cnt_st.at[pl.ds(((c * NS + t) * NS + s) * NPW,
                                             NPW)], s0)
        for t in range(NS):
            pltpu.make_async_copy(
                cnt_loc.at[pl.ds(t * NPW, NPW)],
                cnt_st.at[pl.ds(((c * NS + t) * NS + s) * NPW, NPW)],
                s0).wait()

    plsc.subcore_barrier()

    if compute_cnt:
        pltpu.sync_copy(cnt_st.at[pl.ds((c * NS + s) * NS * NPW, NS * NPW)],
                        cnt_loc)

        def minv(j, _):
            v = cnt_loc[pl.ds(j * LN, LN)]
            for t in range(1, NS):
                v = v + cnt_loc[pl.ds(t * NPW + j * LN, LN)]
            cacc[pl.ds(j * LN, LN)] = 1.0 / jnp.maximum(v, 1.0)
            return 0
        lax.fori_loop(0, NPW // LN, minv, 0)

        @pl.when(c == 0)
        def _():
            pltpu.sync_copy(cacc, inv_out.at[pl.ds(s * NPW, NPW)])
    else:
        pltpu.sync_copy(inv_in.at[pl.ds(s * NPW, NPW)], cacc)

    rbufs = [rbuf, rbuf2]
    ssems = [s0, s1]

    def _oslice(b):
        return agg_out.at[pl.ds(c * NPAD + s * NPW + b * 16, 16)]

    for b in range(NPW // 16):
        p = b % 2
        if b >= 2:
            pltpu.make_async_copy(rbufs[p], _oslice(b - 2), ssems[p]).wait()
        pltpu.sync_copy(acc_sh.at[pl.ds(s * NPW + b * 16, 16)], rbufs[p])

        def scale_row(r, _, _p=p, _b=b):
            iv = plsc.load_gather(cacc, [jnp.full((LN,), _b * 16 + r, _i32)])
            for k in range(D // LN):
                rbufs[_p][r, pl.ds(k * LN, LN)] = (
                    rbufs[_p][r, pl.ds(k * LN, LN)] * iv)
            return 0
        lax.fori_loop(0, 16, scale_row, 0)
        pltpu.async_copy(rbufs[p], _oslice(b), ssems[p])
    for b in range(NPW // 16 - 2, NPW // 16):
        pltpu.make_async_copy(rbufs[b % 2], _oslice(b), ssems[b % 2]).wait()


def _make_seg_mean(compute_cnt):
    mesh = plsc.VectorSubcoreMesh(core_axis_name="c", subcore_axis_name="s")
    out_type = [jax.ShapeDtypeStruct((NC * NPAD, D), _f32)]
    if compute_cnt:
        out_type.append(jax.ShapeDtypeStruct((NPAD,), _f32))
        out_type.append(jax.ShapeDtypeStruct((NC * NS * NPAD,), _f32))
    scratch = [
        pltpu.VMEM((CPB, ECH), _i32),
        pltpu.VMEM((CPB, ECH), _i32),
        pltpu.VMEM((ECH, D), _f32),
        pltpu.VMEM((ECH, D), _f32),
        pltpu.VMEM((16, D), _f32),
        pltpu.VMEM((16, D), _f32),
        pltpu.VMEM((NPAD,), _f32),
        pltpu.VMEM((NPW,), _f32),
        pltpu.SemaphoreType.DMA,
        pltpu.SemaphoreType.DMA,
        pltpu.SemaphoreType.DMA,
        pltpu.SemaphoreType.DMA,
        pltpu.VMEM_SHARED((NPAD, D), _f32),
    ]
    if compute_cnt:
        def body(feat, srcr, dstr, agg_out, inv_out, cnt_st, *rest):
            _seg_mean_body(True, feat, srcr, dstr, None, agg_out, inv_out,
                           cnt_st, *rest)
    else:
        def body(feat, srcr, dstr, inv_in, agg_out, *rest):
            _seg_mean_body(False, feat, srcr, dstr, inv_in, agg_out, None,
                           None, *rest)
    return pl.kernel(body, out_type=tuple(out_type), mesh=mesh,
                     scratch_types=scratch,
                     compiler_params=pltpu.CompilerParams(
                         needs_layout_passes=False))


def _dots_body(h, ia, ib, out, ia0, ia1, ib0, ib1, a0, a1, b0, b1, predv,
               tbuf, sa0, sa1, sb0, sb1, h_sh):
    c = lax.axis_index("c")
    s = lax.axis_index("s")
    w = c * NS + s
    riota = lax.iota(_i32, LN)
    iav = [ia0, ia1]
    ibv = [ib0, ib1]
    abuf = [a0, a1]
    bbuf = [b0, b1]
    sa = [sa0, sa1]
    sb = [sb0, sb1]

    @pl.when(s < NS - 1)
    def _():
        pltpu.sync_copy(h.at[pl.ds(s * 640, 640)], h_sh.at[pl.ds(s * 640, 640)])

    @pl.when(s == NS - 1)
    def _():
        pltpu.sync_copy(h.at[pl.ds((NS - 1) * 640, N - (NS - 1) * 640)],
                        h_sh.at[pl.ds((NS - 1) * 640, N - (NS - 1) * 640)])
    plsc.subcore_barrier()

    def load_and_fire(i, p):
        eb = (w * NLCH + i) * LCH2
        pltpu.sync_copy(ia.at[pl.ds(eb, LCH2)], iav[p])
        pltpu.sync_copy(ib.at[pl.ds(eb, LCH2)], ibv[p])
        pltpu.async_copy(h_sh.at[iav[p]], abuf[p], sa[p])
        pltpu.async_copy(h_sh.at[ibv[p]], bbuf[p], sb[p])

    load_and_fire(0, 0)
    for i in range(NLCH):
        p = i % 2
        if i + 1 < NLCH:
            load_and_fire(i + 1, (i + 1) % 2)
        pltpu.make_async_copy(h_sh.at[iav[p]], abuf[p], sa[p]).wait()
        pltpu.make_async_copy(h_sh.at[ibv[p]], bbuf[p], sb[p]).wait()

        def group(g, _, _p=p):
            def edge2(e2, _):
                for q in range(2):
                    e = g * LN + e2 * 2 + q
                    acc = (abuf[_p][e, pl.ds(0, LN)]
                           * bbuf[_p][e, pl.ds(0, LN)])
                    for k in range(1, D // LN):
                        acc = acc + (abuf[_p][e, pl.ds(k * LN, LN)]
                                     * bbuf[_p][e, pl.ds(k * LN, LN)])
                    tbuf[pl.ds((e2 * 2 + q) * 17, LN)] = acc
                return 0
            lax.fori_loop(0, LN // 2, edge2, 0)

            def colsum(cc, a):
                return a + plsc.load_gather(tbuf, [riota * 17 + cc])
            acc2 = lax.fori_loop(0, LN, colsum, _zero16())
            predv[pl.ds(g * LN, LN)] = acc2
            return 0
        lax.fori_loop(0, LCH2 // LN, group, 0)
        pltpu.sync_copy(predv, out.at[pl.ds((w * NLCH + i) * LCH2, LCH2)])


def _make_dots():
    mesh = plsc.VectorSubcoreMesh(core_axis_name="c", subcore_axis_name="s")
    scratch = [
        pltpu.VMEM((LCH2,), _i32),
        pltpu.VMEM((LCH2,), _i32),
        pltpu.VMEM((LCH2,), _i32),
        pltpu.VMEM((LCH2,), _i32),
        pltpu.VMEM((LCH2, D), _f32),
        pltpu.VMEM((LCH2, D), _f32),
        pltpu.VMEM((LCH2, D), _f32),
        pltpu.VMEM((LCH2, D), _f32),
        pltpu.VMEM((LCH2,), _f32),
        pltpu.VMEM((LN * 17, ), _f32),
        pltpu.SemaphoreType.DMA,
        pltpu.SemaphoreType.DMA,
        pltpu.SemaphoreType.DMA,
        pltpu.SemaphoreType.DMA,
        pltpu.VMEM_SHARED((N, D), _f32),
    ]
    return pl.kernel(_dots_body, out_type=jax.ShapeDtypeStruct((LPAD,), _f32),
                     mesh=mesh, scratch_types=scratch,
                     compiler_params=pltpu.CompilerParams(
                         needs_layout_passes=False))


def _dense_body(x_ref, aggf_ref, wsrc, bsrc, wdst, bdst, wupd, bupd, gam, bet,
                out_ref):
    x = x_ref[...]
    agg = aggf_ref[0:N, :] + aggf_ref[NPAD:NPAD + N, :]
    wu_t = wupd[0:D, :]
    wu_b = wupd[D:2 * D, :]
    hi = jax.lax.Precision.HIGHEST
    w1 = jnp.dot(wdst[...], wu_t, precision=hi)
    w2 = jnp.dot(wsrc[...], wu_b, precision=hi)
    beff = (jnp.dot(bdst[...], wu_t, precision=hi)
            + jnp.dot(bsrc[...], wu_b, precision=hi) + bupd[...])
    h = jnp.dot(x, w1, precision=hi) + jnp.dot(agg, w2, precision=hi) + beff
    m = jnp.mean(h, axis=0, keepdims=True)
    v = jnp.mean(h * h, axis=0, keepdims=True) - m * m
    hn = (h - m) * jax.lax.rsqrt(v + 1.0) * gam[...] + bet[...]
    out_ref[...] = jnp.where(hn >= 0, hn, 0.01 * hn)


def _dense_layer(x, aggf, wsrc, bsrc, wdst, bdst, wupd, bupd, gamma, beta):
    return pl.pallas_call(
        _dense_body,
        out_shape=jax.ShapeDtypeStruct((N, D), _f32),
    )(x, aggf, wsrc, bsrc[None, :], wdst, bdst[None, :], wupd, bupd[None, :],
      gamma[None, :], beta[None, :])


def kernel(x, l1_w_src, l1_b_src, l1_w_dst, l1_b_dst, l1_w_upd, l1_b_upd,
           l2_w_src, l2_b_src, l2_w_dst, l2_b_dst, l2_w_upd, l2_b_upd,
           bn1_gamma, bn1_beta, bn2_gamma, bn2_beta,
           edge_index, edge_label_index):
    E = edge_index.shape[1]
    L = edge_label_index.shape[1]
    src = jnp.concatenate([edge_index[0], jnp.zeros((EPAD - E,), _i32)])
    dst = jnp.concatenate([edge_index[1], jnp.full((EPAD - E,), N, _i32)])
    el0 = jnp.concatenate([edge_label_index[0], jnp.zeros((LPAD - L,), _i32)])
    el1 = jnp.concatenate([edge_label_index[1], jnp.zeros((LPAD - L,), _i32)])
    src = src.reshape(EPAD // ECH, ECH)
    dst = dst.reshape(EPAD // ECH, ECH)

    agg1, inv, _ = _make_seg_mean(True)(x, src, dst)
    h1 = _dense_layer(x, agg1, l1_w_src, l1_b_src, l1_w_dst, l1_b_dst,
                      l1_w_upd, l1_b_upd, bn1_gamma, bn1_beta)
    agg2, = _make_seg_mean(False)(h1, src, dst, inv)
    h2 = _dense_layer(h1, agg2, l2_w_src, l2_b_src, l2_w_dst, l2_b_dst,
                      l2_w_upd, l2_b_upd, bn2_gamma, bn2_beta)
    pred = _make_dots()(h2, el0, el1)
    return pred[:L]
